# Initial kernel scaffold; baseline (speedup 1.0000x reference)
import jax
import jax.numpy as jnp
from jax.experimental import pallas as pl


def kernel(class_number, embedding):
    raise NotImplementedError("write your pallas kernel here")


# speedup vs baseline: 4.1865x; 4.1865x over previous
import functools

import jax
import jax.numpy as jnp
from jax import lax
from jax.experimental import pallas as pl
from jax.experimental.pallas import tpu as pltpu
from jax.experimental.pallas import tpu_sc as plsc

VOCAB_DIM = 64
NUM_B = 16384 * 20
NC, NS = 2, 16
NW = NC * NS
BPW = NUM_B // NW
CHUNK = 512
NCHUNK = BPW // CHUNK


def _make_sc_gather():
    mesh = plsc.VectorSubcoreMesh(
        core_axis_name="c", subcore_axis_name="s", num_cores=NC, num_subcores=NS
    )

    @functools.partial(
        pl.kernel,
        out_type=jax.ShapeDtypeStruct((NUM_B, VOCAB_DIM), jnp.float32),
        mesh=mesh,
        scratch_types=[
            pltpu.VMEM((BPW,), jnp.int32),
            pltpu.VMEM((CHUNK, VOCAB_DIM), jnp.float32),
            pltpu.VMEM((CHUNK, VOCAB_DIM), jnp.float32),
            pltpu.SemaphoreType.DMA,
            pltpu.SemaphoreType.DMA,
        ],
        compiler_params=pltpu.CompilerParams(use_tc_tiling_on_sc=False),
    )
    def gather_kernel(table_hbm, idx_hbm, out_hbm, idx_v, buf0, buf1, sem0, sem1):
        wid = lax.axis_index("s") * NC + lax.axis_index("c")
        base = wid * BPW
        pltpu.sync_copy(idx_hbm.at[pl.ds(base, BPW)], idx_v)

        bufs = (buf0, buf1)
        sems = (sem0, sem1)

        pltpu.async_copy(table_hbm.at[idx_v.at[pl.ds(0, CHUNK)]], buf0, sem0)

        @pl.loop(0, NCHUNK, step=2)
        def _(c):
            for j in range(2):
                buf, sem = bufs[j], sems[j]
                nbuf, nsem = bufs[1 - j], sems[1 - j]
                step = c + j
                @pl.when(step + 1 < NCHUNK)
                def _():
                    off = (step + 1) * CHUNK
                    pltpu.async_copy(
                        table_hbm.at[idx_v.at[pl.ds(off, CHUNK)]], nbuf, nsem
                    )

                pltpu.make_async_copy(
                    table_hbm.at[idx_v.at[pl.ds(0, CHUNK)]], buf, sem
                ).wait()
                pltpu.sync_copy(buf, out_hbm.at[pl.ds(base + step * CHUNK, CHUNK)])

    return gather_kernel


_sc_gather = _make_sc_gather()


@jax.jit
def kernel(class_number, embedding):
    idx = class_number.reshape(-1).astype(jnp.int32)
    flat = _sc_gather(embedding, idx)
    return flat.reshape(class_number.shape + (VOCAB_DIM,))
